# baseline (speedup 1.0000x reference)
import functools

import jax
import jax.numpy as jnp
from jax.experimental import pallas as pl
from jax.experimental.pallas import tpu as pltpu

N_TOK = 4096
D = 32
K = 8192
BN = 512


def _norm_body(feat_ref, cb_ref, fn_ref, cn_ref):
    f = feat_ref[...]
    fn_ref[...] = f / (jnp.sqrt(jnp.sum(f * f, axis=1, keepdims=True)) + 1e-12)
    c = cb_ref[...]
    cn_ref[...] = c / (jnp.sqrt(jnp.sum(c * c, axis=1, keepdims=True)) + 1e-12)


def _mm_into(fn_ref, cn_ref, s_ref):
    s_ref[...] = jax.lax.dot_general(
        fn_ref[...], cn_ref[...],
        dimension_numbers=(((1,), (1,)), ((), ())),
        preferred_element_type=jnp.float32)


def _tail_from(s_ref, ki_ref, idx_ref):
    d = 1.0 - s_ref[...]
    dmin = jnp.min(d, axis=1, keepdims=True)
    midx = jnp.min(
        jnp.where(d == dmin, ki_ref[...], jnp.int32(2**31 - 1)),
        axis=1, keepdims=True)
    idx_ref[...] = midx


def _argmin_body(fn_ref, cn_ref, ki_ref, idx_ref, s0_ref, s1_ref):
    n = pl.program_id(0)

    @pl.when(n % 2 == 0)
    def _even():
        _mm_into(fn_ref, cn_ref, s0_ref)
        _tail_from(s1_ref, ki_ref, idx_ref)

    @pl.when(n % 2 == 1)
    def _odd():
        _mm_into(fn_ref, cn_ref, s1_ref)
        _tail_from(s0_ref, ki_ref, idx_ref)


def _nearest_idx(z, codebook):
    feat = z.reshape(N_TOK, D)
    fn, cn = pl.pallas_call(
        _norm_body,
        grid=(1,),
        in_specs=[
            pl.BlockSpec((N_TOK, D), lambda i: (0, 0)),
            pl.BlockSpec((K, D), lambda i: (0, 0)),
        ],
        out_specs=[
            pl.BlockSpec((N_TOK, D), lambda i: (0, 0)),
            pl.BlockSpec((K, D), lambda i: (0, 0)),
        ],
        out_shape=[
            jax.ShapeDtypeStruct((N_TOK, D), jnp.float32),
            jax.ShapeDtypeStruct((K, D), jnp.float32),
        ],
    )(feat, codebook)
    ki = jax.lax.broadcasted_iota(jnp.int32, (1, K), 1)
    nb = N_TOK // BN
    idx2 = pl.pallas_call(
        _argmin_body,
        grid=(nb + 1,),
        in_specs=[
            pl.BlockSpec((BN, D), lambda n: (jnp.minimum(n, nb - 1), 0)),
            pl.BlockSpec((K, D), lambda n: (0, 0)),
            pl.BlockSpec((1, K), lambda n: (0, 0)),
        ],
        out_specs=pl.BlockSpec((BN, 1), lambda n: (jnp.maximum(n - 1, 0), 0)),
        out_shape=jax.ShapeDtypeStruct((N_TOK, 1), jnp.int32),
        scratch_shapes=[pltpu.VMEM((BN, K), jnp.float32),
                        pltpu.VMEM((BN, K), jnp.float32)],
    )(fn, cn, ki)
    return idx2.reshape(N_TOK)


def kernel(z, codebook, cluster_labels):
    bs = z.shape[0]
    idx = _nearest_idx(z, codebook)
    token_labels = jnp.take(cluster_labels, idx, axis=0)
    img = token_labels.reshape(bs, 1, 16, 16).astype(jnp.float32)
    out = jnp.repeat(jnp.repeat(img, 14, axis=2), 14, axis=3)
    return out


# speedup vs baseline: 1.5899x; 1.5899x over previous
import functools

import jax
import jax.numpy as jnp
from jax.experimental import pallas as pl
from jax.experimental.pallas import tpu as pltpu
from jax.experimental.pallas import tpu_sc as plsc

N_TOK = 4096
D = 32
K = 8192
BN = 1024


def _argmin_body(feat_ref, cb_ref, idx_ref, cn_ref):
    @pl.when(pl.program_id(0) == 0)
    def _prep():
        c = cb_ref[...]
        cn_ref[...] = c / (
            jnp.sqrt(jnp.sum(c * c, axis=1, keepdims=True)) + 1e-12)

    f = feat_ref[...]
    fn = f / (jnp.sqrt(jnp.sum(f * f, axis=1, keepdims=True)) + 1e-12)
    s = jax.lax.dot_general(
        fn, cn_ref[...],
        dimension_numbers=(((1,), (1,)), ((), ())),
        preferred_element_type=jnp.float32)
    d = 1.0 - s
    midx = jnp.argmin(d, axis=1).astype(jnp.int32)
    idx_ref[...] = midx[:, None]


def _nearest_idx(z, codebook):
    feat = z.reshape(N_TOK, D)
    nb = N_TOK // BN
    idx2 = pl.pallas_call(
        _argmin_body,
        grid=(nb,),
        in_specs=[
            pl.BlockSpec((BN, D), lambda n: (n, 0)),
            pl.BlockSpec((K, D), lambda n: (0, 0)),
        ],
        out_specs=pl.BlockSpec((BN, 1), lambda n: (n, 0)),
        out_shape=jax.ShapeDtypeStruct((N_TOK, 1), jnp.int32),
        scratch_shapes=[pltpu.VMEM((K, D), jnp.float32)],
    )(feat, codebook)
    return idx2


NW = 32
TPW = N_TOK // NW


def _upsample_sc(idx, cluster_labels, exp_idx):
    mesh = plsc.VectorSubcoreMesh(core_axis_name="c", subcore_axis_name="s")

    @functools.partial(
        pl.kernel,
        mesh=mesh,
        out_type=jax.ShapeDtypeStruct((16, 1, 224, 224), jnp.float32),
        compiler_params=pltpu.CompilerParams(needs_layout_passes=False),
        scratch_types=[
            pltpu.VMEM((TPW, 1), jnp.int32),
            pltpu.VMEM((TPW,), jnp.int32),
            pltpu.VMEM((224,), jnp.int32),
            pltpu.VMEM((TPW,), jnp.int32),
            pltpu.VMEM((112, 224), jnp.float32),
            pltpu.SemaphoreType.DMA,
            pltpu.SemaphoreType.DMA,
        ],
    )
    def sc_body(idx_hbm, lab_hbm, exp_hbm, out_hbm, idx2_v, idx_v, exp_v,
                labi_v, rows_v, sem0, sem1):
        wid = jax.lax.axis_index("s") * 2 + jax.lax.axis_index("c")
        cp_exp = pltpu.async_copy(exp_hbm, exp_v, sem1)
        pltpu.async_copy(idx_hbm.at[pl.ds(wid * TPW, TPW), :], idx2_v,
                         sem0).wait()
        lane = jax.lax.broadcasted_iota(jnp.int32, (16,), 0)
        zero = lane * 0
        for t in range(TPW // 16):
            iv = plsc.load_gather(idx2_v, [lane + 16 * t, zero])
            idx_v[pl.ds(16 * t, 16)] = iv
        pltpu.async_copy(lab_hbm.at[idx_v], labi_v, sem0).wait()
        cp_exp.wait()
        for pr in range(8):
            for v in range(14):
                xi = exp_v[pl.ds(16 * v, 16)] + 16 * pr
                seg = plsc.load_gather(labi_v, [xi]).astype(jnp.float32)
                for j in range(14):
                    rows_v[pr * 14 + j, pl.ds(16 * v, 16)] = seg
        pltpu.sync_copy(
            rows_v,
            out_hbm.at[wid // 2, 0, pl.ds((wid % 2) * 112, 112), :])

    return sc_body(idx, cluster_labels, exp_idx)


def kernel(z, codebook, cluster_labels):
    idx = _nearest_idx(z, codebook)
    exp_idx = jnp.arange(224, dtype=jnp.int32) // 14
    return _upsample_sc(idx, cluster_labels, exp_idx)
